# Initial kernel scaffold; baseline (speedup 1.0000x reference)
import jax
import jax.numpy as jnp
from jax.experimental import pallas as pl


def kernel(nodes, senders, receivers, W1, b1, W2, b2):
    raise NotImplementedError("write your pallas kernel here")


# speedup vs baseline: 18.9901x; 18.9901x over previous
import functools
import math

import jax
import jax.numpy as jnp
from jax import lax
from jax.experimental import pallas as pl
from jax.experimental.pallas import tpu as pltpu
from jax.experimental.pallas import tpu_sc as plsc

N_NODES = 10000
D_FEAT = 128
NUM_HEADS = 4
FEATS_PER_HEAD = 32

N_PAD = 10240
NW = 32
W = 128
INV_SQRT_F = 1.0 / math.sqrt(float(FEATS_PER_HEAD))


def _mlp_body(x_ref, w1_ref, b1_ref, w2_ref, b2_ref, out_ref):
    h = jnp.dot(x_ref[...], w1_ref[...], preferred_element_type=jnp.float32)
    h = jnp.maximum(h + b1_ref[...], 0.0)
    h = jnp.dot(h, w2_ref[...], preferred_element_type=jnp.float32)
    out_ref[...] = jnp.maximum(h + b2_ref[...], 0.0)


def _logits_body(sent_ref, recv_ref, sel_ref, logit_ref, bmax_ref):
    p = sent_ref[...] * recv_ref[...]
    logits = jnp.dot(p, sel_ref[...], preferred_element_type=jnp.float32)
    logits = logits * INV_SQRT_F
    logit_ref[...] = logits
    bmax_ref[...] = jnp.max(logits, axis=0, keepdims=True)[None]


def _exp_body(logit_ref, bmax_ref, ex_ref):
    gmax = jnp.max(jnp.squeeze(bmax_ref[...], axis=1), axis=0, keepdims=True)
    ex_ref[...] = jnp.exp(logit_ref[...] - gmax)


def _msg_body(sent_ref, ex_ref, den_ref, selt_ref, msg_ref):
    w = ex_ref[...] / den_ref[...]
    wexp = jnp.dot(w, selt_ref[...], preferred_element_type=jnp.float32)
    msg_ref[...] = sent_ref[...] * wexp


def _final_body(p0_ref, p1_ref, out_ref):
    out_ref[...] = jnp.maximum(p0_ref[...] + p1_ref[...], 0.0)


def _gather_body(emb, senders, receivers, sent_out, recv_out,
                 sidx_v, ridx_v, sent_v, recv_v, sem_s, sem_r):
    e_pad = senders.shape[0]
    per_w = e_pad // NW
    wid = lax.axis_index("s") * 2 + lax.axis_index("c")
    base = wid * per_w

    def body(w, _):
        off = base + w * W
        pltpu.sync_copy(senders.at[pl.ds(off, W)], sidx_v)
        pltpu.sync_copy(receivers.at[pl.ds(off, W)], ridx_v)
        cp_s = pltpu.async_copy(emb.at[sidx_v], sent_v, sem_s)
        cp_r = pltpu.async_copy(emb.at[ridx_v], recv_v, sem_r)
        cp_s.wait()
        cp_r.wait()
        pltpu.sync_copy(sent_v, sent_out.at[pl.ds(off, W)])
        pltpu.sync_copy(recv_v, recv_out.at[pl.ds(off, W)])
        return ()

    lax.fori_loop(0, per_w // W, body, (), unroll=False)


def _segsum_body(receivers, ex, zeros4, den_out,
                 ridx_v, exw_v, den_v, seg_tbl, sem_g):
    e_pad = receivers.shape[0]
    core = lax.axis_index("c")
    sid = lax.axis_index("s")
    per_w = e_pad // 16
    base = sid * per_w
    rows = N_PAD // 16

    @pl.when(core == 0)
    def _zero():
        pltpu.sync_copy(zeros4.at[pl.ds(sid * rows, rows)],
                        seg_tbl.at[pl.ds(sid * rows, rows)])

    plsc.subcore_barrier()

    @pl.when(core == 0)
    def _scatter():
        def body(w, _):
            off = base + w * W
            pltpu.sync_copy(receivers.at[pl.ds(off, W)], ridx_v)
            pltpu.sync_copy(ex.at[pl.ds(off, W)], exw_v)
            pltpu.async_copy(exw_v, seg_tbl.at[ridx_v], sem_g,
                             add=True).wait()
            return ()
        lax.fori_loop(0, per_w // W, body, (), unroll=False)

    plsc.subcore_barrier()

    @pl.when(core == 0)
    def _gather_den():
        def body(w, _):
            off = base + w * W
            pltpu.sync_copy(receivers.at[pl.ds(off, W)], ridx_v)
            pltpu.async_copy(seg_tbl.at[ridx_v], den_v, sem_g).wait()
            pltpu.sync_copy(den_v, den_out.at[pl.ds(off, W)])
            return ()
        lax.fori_loop(0, per_w // W, body, (), unroll=False)


def _scatter_msg_body(receivers, msg, zeros128, part_out,
                      ridx_v, msg_v, agg_tbl, sem_g):
    e_pad = receivers.shape[0]
    core = lax.axis_index("c")
    sid = lax.axis_index("s")
    half = e_pad // 2
    per_w = half // 16
    base = core * half + sid * per_w
    rows = N_PAD // 16

    pltpu.sync_copy(zeros128.at[pl.ds(sid * rows, rows)],
                    agg_tbl.at[pl.ds(sid * rows, rows)])
    plsc.subcore_barrier()

    def body(w, _):
        off = base + w * W
        pltpu.sync_copy(receivers.at[pl.ds(off, W)], ridx_v)
        pltpu.sync_copy(msg.at[pl.ds(off, W)], msg_v)
        pltpu.async_copy(msg_v, agg_tbl.at[ridx_v], sem_g, add=True).wait()
        return ()

    lax.fori_loop(0, per_w // W, body, (), unroll=False)

    plsc.subcore_barrier()
    pltpu.sync_copy(agg_tbl.at[pl.ds(sid * rows, rows)],
                    part_out.at[core, pl.ds(sid * rows, rows)])


def kernel(nodes, senders, receivers, W1, b1, W2, b2):
    n = nodes.shape[0]
    e = senders.shape[0]
    chunk = NW * W * 10
    e_pad = ((e + chunk - 1) // chunk) * chunk
    npad_edges = e_pad - e

    nodes_p = jnp.concatenate(
        [nodes, jnp.zeros((N_PAD - n, D_FEAT), jnp.float32)])
    senders_p = jnp.concatenate(
        [senders, jnp.zeros((npad_edges,), jnp.int32)])
    dummy = n + (jnp.arange(npad_edges, dtype=jnp.int32) % (N_PAD - n))
    receivers_p = jnp.concatenate([receivers, dummy])

    sel = jnp.repeat(jnp.eye(NUM_HEADS, dtype=jnp.float32),
                     FEATS_PER_HEAD, axis=0)
    selt = sel.T

    nblk = 8
    brows = N_PAD // nblk
    emb = pl.pallas_call(
        _mlp_body,
        grid=(nblk,),
        in_specs=[
            pl.BlockSpec((brows, D_FEAT), lambda i: (i, 0)),
            pl.BlockSpec((D_FEAT, D_FEAT), lambda i: (0, 0)),
            pl.BlockSpec((1, D_FEAT), lambda i: (0, 0)),
            pl.BlockSpec((D_FEAT, D_FEAT), lambda i: (0, 0)),
            pl.BlockSpec((1, D_FEAT), lambda i: (0, 0)),
        ],
        out_specs=pl.BlockSpec((brows, D_FEAT), lambda i: (i, 0)),
        out_shape=jax.ShapeDtypeStruct((N_PAD, D_FEAT), jnp.float32),
    )(nodes_p, W1, b1.reshape(1, -1), W2, b2.reshape(1, -1))

    mesh = plsc.VectorSubcoreMesh(core_axis_name="c", subcore_axis_name="s")

    sent, recv = pl.kernel(
        _gather_body,
        out_type=[jax.ShapeDtypeStruct((e_pad, D_FEAT), jnp.float32),
                  jax.ShapeDtypeStruct((e_pad, D_FEAT), jnp.float32)],
        mesh=mesh,
        scratch_types=[
            pltpu.VMEM((W,), jnp.int32),
            pltpu.VMEM((W,), jnp.int32),
            pltpu.VMEM((W, D_FEAT), jnp.float32),
            pltpu.VMEM((W, D_FEAT), jnp.float32),
            pltpu.SemaphoreType.DMA,
            pltpu.SemaphoreType.DMA,
        ],
    )(emb, senders_p, receivers_p)

    be = 2048
    ne_blk = e_pad // be
    logits, bmax = pl.pallas_call(
        _logits_body,
        grid=(ne_blk,),
        in_specs=[
            pl.BlockSpec((be, D_FEAT), lambda i: (i, 0)),
            pl.BlockSpec((be, D_FEAT), lambda i: (i, 0)),
            pl.BlockSpec((D_FEAT, NUM_HEADS), lambda i: (0, 0)),
        ],
        out_specs=[
            pl.BlockSpec((be, NUM_HEADS), lambda i: (i, 0)),
            pl.BlockSpec((1, 1, NUM_HEADS), lambda i: (i, 0, 0)),
        ],
        out_shape=[jax.ShapeDtypeStruct((e_pad, NUM_HEADS), jnp.float32),
                   jax.ShapeDtypeStruct((ne_blk, 1, NUM_HEADS), jnp.float32)],
    )(sent, recv, sel)

    ex = pl.pallas_call(
        _exp_body,
        grid=(ne_blk,),
        in_specs=[
            pl.BlockSpec((be, NUM_HEADS), lambda i: (i, 0)),
            pl.BlockSpec((ne_blk, 1, NUM_HEADS), lambda i: (0, 0, 0)),
        ],
        out_specs=pl.BlockSpec((be, NUM_HEADS), lambda i: (i, 0)),
        out_shape=jax.ShapeDtypeStruct((e_pad, NUM_HEADS), jnp.float32),
    )(logits, bmax)

    zeros4 = jnp.zeros((N_PAD, NUM_HEADS), jnp.float32)
    den = pl.kernel(
        _segsum_body,
        out_type=jax.ShapeDtypeStruct((e_pad, NUM_HEADS), jnp.float32),
        mesh=mesh,
        scratch_types=[
            pltpu.VMEM((W,), jnp.int32),
            pltpu.VMEM((W, NUM_HEADS), jnp.float32),
            pltpu.VMEM((W, NUM_HEADS), jnp.float32),
            pltpu.VMEM_SHARED((N_PAD, NUM_HEADS), jnp.float32),
            pltpu.SemaphoreType.DMA,
        ],
    )(receivers_p, ex, zeros4, )

    msg = pl.pallas_call(
        _msg_body,
        grid=(ne_blk,),
        in_specs=[
            pl.BlockSpec((be, D_FEAT), lambda i: (i, 0)),
            pl.BlockSpec((be, NUM_HEADS), lambda i: (i, 0)),
            pl.BlockSpec((be, NUM_HEADS), lambda i: (i, 0)),
            pl.BlockSpec((NUM_HEADS, D_FEAT), lambda i: (0, 0)),
        ],
        out_specs=pl.BlockSpec((be, D_FEAT), lambda i: (i, 0)),
        out_shape=jax.ShapeDtypeStruct((e_pad, D_FEAT), jnp.float32),
    )(sent, ex, den, selt)

    zeros128 = jnp.zeros((N_PAD, D_FEAT), jnp.float32)
    part = pl.kernel(
        _scatter_msg_body,
        out_type=jax.ShapeDtypeStruct((2, N_PAD, D_FEAT), jnp.float32),
        mesh=mesh,
        scratch_types=[
            pltpu.VMEM((W,), jnp.int32),
            pltpu.VMEM((W, D_FEAT), jnp.float32),
            pltpu.VMEM_SHARED((N_PAD, D_FEAT), jnp.float32),
            pltpu.SemaphoreType.DMA,
        ],
    )(receivers_p, msg, zeros128)

    nfb = 10
    frows = n // nfb
    out = pl.pallas_call(
        _final_body,
        grid=(nfb,),
        in_specs=[
            pl.BlockSpec((frows, D_FEAT), lambda i: (i, 0)),
            pl.BlockSpec((frows, D_FEAT), lambda i: (i, 0)),
        ],
        out_specs=pl.BlockSpec((frows, D_FEAT), lambda i: (i, 0)),
        out_shape=jax.ShapeDtypeStruct((n, D_FEAT), jnp.float32),
    )(part[0], part[1])

    return out


# speedup vs baseline: 21.7631x; 1.1460x over previous
import math

import jax
import jax.numpy as jnp
from jax import lax
from jax.experimental import pallas as pl
from jax.experimental.pallas import tpu as pltpu
from jax.experimental.pallas import tpu_sc as plsc

D_FEAT = 128
NUM_HEADS = 4
FEATS_PER_HEAD = 32

N_PAD = 10240
NW = 32
W = 128
INV_SQRT_F = 1.0 / math.sqrt(float(FEATS_PER_HEAD))


def _mlp_body(x_ref, w1_ref, b1_ref, w2_ref, b2_ref, out_ref):
    h = jnp.dot(x_ref[...], w1_ref[...], preferred_element_type=jnp.float32)
    h = jnp.maximum(h + b1_ref[...], 0.0)
    h = jnp.dot(h, w2_ref[...], preferred_element_type=jnp.float32)
    out_ref[...] = jnp.maximum(h + b2_ref[...], 0.0)


def _logits_body(sent_ref, recv_ref, sel_ref, logit_ref, bmax_ref):
    p = sent_ref[...] * recv_ref[...]
    logits = jnp.dot(p, sel_ref[...], preferred_element_type=jnp.float32)
    logits = logits * INV_SQRT_F
    logit_ref[...] = logits
    bmax_ref[...] = jnp.max(logits, axis=0, keepdims=True)[None]


def _exp_body(logit_ref, bmax_ref, ex_ref):
    gmax = jnp.max(jnp.squeeze(bmax_ref[...], axis=1), axis=0, keepdims=True)
    ex_ref[...] = jnp.exp(logit_ref[...] - gmax)


def _msg_body(sent_ref, ex_ref, selt_ref, msg_ref, wexp_ref):
    wexp = jnp.dot(ex_ref[...], selt_ref[...],
                   preferred_element_type=jnp.float32)
    wexp_ref[...] = wexp
    msg_ref[...] = sent_ref[...] * wexp


def _final_body(p0_ref, p1_ref, s0_ref, s1_ref, out_ref):
    d = s0_ref[...] + s1_ref[...]
    d = jnp.where(d > 0.0, d, 1.0)
    out_ref[...] = jnp.maximum((p0_ref[...] + p1_ref[...]) / d, 0.0)


def _gather_body(emb, senders, receivers, sent_out, recv_out,
                 sidx, ridx, sbuf, rbuf,
                 gs0, gs1, gr0, gr1, ws0, ws1, wr0, wr1):
    n_edge = senders.shape[0]
    per_w = n_edge // NW
    nwin = per_w // W
    wid = lax.axis_index("s") * 2 + lax.axis_index("c")
    base = wid * per_w

    def body(w, _):
        off = base + w * W
        pltpu.sync_copy(senders.at[pl.ds(off, W)], sidx)
        pltpu.sync_copy(receivers.at[pl.ds(off, W)], ridx)
        cp_s = pltpu.async_copy(emb.at[sidx], sbuf.at[0], gs0)
        cp_r = pltpu.async_copy(emb.at[ridx], rbuf.at[0], gr0)
        cp_s.wait()
        cp_r.wait()
        pltpu.sync_copy(sbuf.at[0], sent_out.at[pl.ds(off, W)])
        pltpu.sync_copy(rbuf.at[0], recv_out.at[pl.ds(off, W)])
        return ()

    lax.fori_loop(0, nwin, body, (), unroll=False)


def _scatter_msg_body(receivers, msg2d, zeros128, part_out,
                      i0, i1, mbuf, agg_tbl, il0, il1, ls0, ls1, ss0, ss1):
    n_edge = receivers.shape[0]
    per_w = n_edge // NW
    nwin = per_w // W
    core = lax.axis_index("c")
    sid = lax.axis_index("s")
    base = core * (n_edge // 2) + sid * per_w
    rows = N_PAD // 16

    pltpu.sync_copy(zeros128.at[pl.ds(sid * rows, rows)],
                    agg_tbl.at[pl.ds(sid * rows, rows)])
    plsc.subcore_barrier()

    def body(w, _):
        off = base + w * W
        pltpu.sync_copy(receivers.at[pl.ds(off, W)], i0)
        pltpu.sync_copy(msg2d.at[pl.ds(off, W)], mbuf.at[0])
        pltpu.async_copy(mbuf.at[0], agg_tbl.at[i0], ss0, add=True).wait()
        return ()

    lax.fori_loop(0, nwin, body, (), unroll=False)

    plsc.subcore_barrier()
    pltpu.sync_copy(agg_tbl.at[pl.ds(sid * rows, rows)],
                    part_out.at[core, pl.ds(sid * rows, rows)])


def kernel(nodes, senders, receivers, W1, b1, W2, b2):
    n = nodes.shape[0]
    e = senders.shape[0]
    chunk = NW * W * 10
    e_pad = ((e + chunk - 1) // chunk) * chunk
    npad_edges = e_pad - e

    nodes_p = jnp.concatenate(
        [nodes, jnp.zeros((N_PAD - n, D_FEAT), jnp.float32)])
    senders_p = jnp.concatenate(
        [senders, jnp.zeros((npad_edges,), jnp.int32)])
    dummy = n + (jnp.arange(npad_edges, dtype=jnp.int32) % (N_PAD - n))
    receivers_p = jnp.concatenate([receivers, dummy])

    sel = jnp.repeat(jnp.eye(NUM_HEADS, dtype=jnp.float32),
                     FEATS_PER_HEAD, axis=0)
    selt = sel.T

    nblk = 8
    brows = N_PAD // nblk
    emb = pl.pallas_call(
        _mlp_body,
        grid=(nblk,),
        in_specs=[
            pl.BlockSpec((brows, D_FEAT), lambda i: (i, 0)),
            pl.BlockSpec((D_FEAT, D_FEAT), lambda i: (0, 0)),
            pl.BlockSpec((1, D_FEAT), lambda i: (0, 0)),
            pl.BlockSpec((D_FEAT, D_FEAT), lambda i: (0, 0)),
            pl.BlockSpec((1, D_FEAT), lambda i: (0, 0)),
        ],
        out_specs=pl.BlockSpec((brows, D_FEAT), lambda i: (i, 0)),
        out_shape=jax.ShapeDtypeStruct((N_PAD, D_FEAT), jnp.float32),
    )(nodes_p, W1, b1.reshape(1, -1), W2, b2.reshape(1, -1))

    mesh = plsc.VectorSubcoreMesh(core_axis_name="c", subcore_axis_name="s")
    per_w = e_pad // NW

    sent, recv = pl.kernel(
        _gather_body,
        out_type=[jax.ShapeDtypeStruct((e_pad, D_FEAT), jnp.float32),
                  jax.ShapeDtypeStruct((e_pad, D_FEAT), jnp.float32)],
        mesh=mesh,
        scratch_types=[
            pltpu.VMEM((W,), jnp.int32),
            pltpu.VMEM((W,), jnp.int32),
            pltpu.VMEM((2, W, D_FEAT), jnp.float32),
            pltpu.VMEM((2, W, D_FEAT), jnp.float32),
        ] + [pltpu.SemaphoreType.DMA] * 8,
    )(emb, senders_p, receivers_p)

    be = 2048
    ne_blk = e_pad // be
    logits, bmax = pl.pallas_call(
        _logits_body,
        grid=(ne_blk,),
        in_specs=[
            pl.BlockSpec((be, D_FEAT), lambda i: (i, 0)),
            pl.BlockSpec((be, D_FEAT), lambda i: (i, 0)),
            pl.BlockSpec((D_FEAT, NUM_HEADS), lambda i: (0, 0)),
        ],
        out_specs=[
            pl.BlockSpec((be, NUM_HEADS), lambda i: (i, 0)),
            pl.BlockSpec((1, 1, NUM_HEADS), lambda i: (i, 0, 0)),
        ],
        out_shape=[jax.ShapeDtypeStruct((e_pad, NUM_HEADS), jnp.float32),
                   jax.ShapeDtypeStruct((ne_blk, 1, NUM_HEADS), jnp.float32)],
    )(sent, recv, sel)

    ex = pl.pallas_call(
        _exp_body,
        grid=(ne_blk,),
        in_specs=[
            pl.BlockSpec((be, NUM_HEADS), lambda i: (i, 0)),
            pl.BlockSpec((ne_blk, 1, NUM_HEADS), lambda i: (0, 0, 0)),
        ],
        out_specs=pl.BlockSpec((be, NUM_HEADS), lambda i: (i, 0)),
        out_shape=jax.ShapeDtypeStruct((e_pad, NUM_HEADS), jnp.float32),
    )(logits, bmax)

    msg = pl.pallas_call(
        _msg_body,
        grid=(ne_blk,),
        in_specs=[
            pl.BlockSpec((be, D_FEAT), lambda i: (i, 0)),
            pl.BlockSpec((be, NUM_HEADS), lambda i: (i, 0)),
            pl.BlockSpec((NUM_HEADS, D_FEAT), lambda i: (0, 0)),
        ],
        out_specs=[pl.BlockSpec((be, D_FEAT), lambda i: (i, 0)),
                   pl.BlockSpec((be, D_FEAT), lambda i: (i, 0))],
        out_shape=[jax.ShapeDtypeStruct((e_pad, D_FEAT), jnp.float32),
                   jax.ShapeDtypeStruct((e_pad, D_FEAT), jnp.float32)],
    )(sent, ex, selt)
    msg, wexp = msg

    zeros128 = jnp.zeros((N_PAD, D_FEAT), jnp.float32)
    part = pl.kernel(
        _scatter_msg_body,
        out_type=jax.ShapeDtypeStruct((2, N_PAD, D_FEAT), jnp.float32),
        mesh=mesh,
        scratch_types=[
            pltpu.VMEM((W,), jnp.int32),
            pltpu.VMEM((W,), jnp.int32),
            pltpu.VMEM((2, W, D_FEAT), jnp.float32),
            pltpu.VMEM_SHARED((N_PAD, D_FEAT), jnp.float32),
        ] + [pltpu.SemaphoreType.DMA] * 6,
    )(receivers_p, msg, zeros128)

    segp = pl.kernel(
        _scatter_msg_body,
        out_type=jax.ShapeDtypeStruct((2, N_PAD, D_FEAT), jnp.float32),
        mesh=mesh,
        scratch_types=[
            pltpu.VMEM((W,), jnp.int32),
            pltpu.VMEM((W,), jnp.int32),
            pltpu.VMEM((2, W, D_FEAT), jnp.float32),
            pltpu.VMEM_SHARED((N_PAD, D_FEAT), jnp.float32),
        ] + [pltpu.SemaphoreType.DMA] * 6,
    )(receivers_p, wexp, zeros128)

    nfb = 10
    frows = n // nfb
    out = pl.pallas_call(
        _final_body,
        grid=(nfb,),
        in_specs=[
            pl.BlockSpec((frows, D_FEAT), lambda i: (i, 0)),
            pl.BlockSpec((frows, D_FEAT), lambda i: (i, 0)),
            pl.BlockSpec((frows, D_FEAT), lambda i: (i, 0)),
            pl.BlockSpec((frows, D_FEAT), lambda i: (i, 0)),
        ],
        out_specs=pl.BlockSpec((frows, D_FEAT), lambda i: (i, 0)),
        out_shape=jax.ShapeDtypeStruct((n, D_FEAT), jnp.float32),
    )(part[0], part[1], segp[0], segp[1])

    return out


# speedup vs baseline: 26.4603x; 1.2158x over previous
import math

import jax
import jax.numpy as jnp
from jax import lax
from jax.experimental import pallas as pl
from jax.experimental.pallas import tpu as pltpu
from jax.experimental.pallas import tpu_sc as plsc

D_FEAT = 128
NUM_HEADS = 4
FEATS_PER_HEAD = 32

N_PAD = 10240
NW = 32
W = 128
INV_SQRT_F = 1.0 / math.sqrt(float(FEATS_PER_HEAD))


def _mlp_body(x_ref, w1_ref, b1_ref, w2_ref, b2_ref, out_ref):
    h = jnp.dot(x_ref[...], w1_ref[...], preferred_element_type=jnp.float32)
    h = jnp.maximum(h + b1_ref[...], 0.0)
    h = jnp.dot(h, w2_ref[...], preferred_element_type=jnp.float32)
    out_ref[...] = jnp.maximum(h + b2_ref[...], 0.0)


def _logits_body(sent_ref, recv_ref, sel_ref, logit_ref, bmax_ref):
    p = sent_ref[...] * recv_ref[...]
    logits = jnp.dot(p, sel_ref[...], preferred_element_type=jnp.float32)
    logits = logits * INV_SQRT_F
    logit_ref[...] = logits
    bmax_ref[...] = jnp.max(logits, axis=0, keepdims=True)[None]


def _exp_body(logit_ref, bmax_ref, ex_ref):
    gmax = jnp.max(jnp.squeeze(bmax_ref[...], axis=1), axis=0, keepdims=True)
    ex_ref[...] = jnp.exp(logit_ref[...] - gmax)


def _msg_body(sent_ref, ex_ref, selt_ref, msg_ref, wexp_ref):
    wexp = jnp.dot(ex_ref[...], selt_ref[...],
                   preferred_element_type=jnp.float32)
    wexp_ref[...] = wexp
    msg_ref[...] = sent_ref[...] * wexp


def _final_body(p0_ref, p1_ref, s0_ref, s1_ref, out_ref):
    d = s0_ref[...] + s1_ref[...]
    d = jnp.where(d > 0.0, d, 1.0)
    out_ref[...] = jnp.maximum((p0_ref[...] + p1_ref[...]) / d, 0.0)


def _gather_body(emb, senders, receivers, sent_out, recv_out,
                 sidx, ridx, sbuf, rbuf,
                 gs0, gs1, gr0, gr1, ws0, ws1, wr0, wr1):
    n_edge = senders.shape[0]
    per_w = n_edge // NW
    nwin = per_w // W
    wid = lax.axis_index("s") * 2 + lax.axis_index("c")
    base = wid * per_w
    pltpu.sync_copy(senders.at[pl.ds(base, per_w)], sidx)
    pltpu.sync_copy(receivers.at[pl.ds(base, per_w)], ridx)

    gssem = (gs0, gs1)
    grsem = (gr0, gr1)
    wssem = (ws0, ws1)
    wrsem = (wr0, wr1)

    def gstart(w, b):
        pltpu.async_copy(emb.at[sidx.at[pl.ds(w * W, W)]], sbuf.at[b],
                         gssem[b])
        pltpu.async_copy(emb.at[ridx.at[pl.ds(w * W, W)]], rbuf.at[b],
                         grsem[b])

    def gwait(w, b):
        pltpu.make_async_copy(emb.at[sidx.at[pl.ds(w * W, W)]], sbuf.at[b],
                              gssem[b]).wait()
        pltpu.make_async_copy(emb.at[ridx.at[pl.ds(w * W, W)]], rbuf.at[b],
                              grsem[b]).wait()

    def wstart(w, b):
        off = base + w * W
        pltpu.async_copy(sbuf.at[b], sent_out.at[pl.ds(off, W)], wssem[b])
        pltpu.async_copy(rbuf.at[b], recv_out.at[pl.ds(off, W)], wrsem[b])

    def wwait(w, b):
        off = base + w * W
        pltpu.make_async_copy(sbuf.at[b], sent_out.at[pl.ds(off, W)],
                              wssem[b]).wait()
        pltpu.make_async_copy(rbuf.at[b], recv_out.at[pl.ds(off, W)],
                              wrsem[b]).wait()

    gstart(0, 0)
    gwait(0, 0)
    gstart(1, 1)
    wstart(0, 0)

    def body(g, _):
        w1 = 2 * g + 1
        gwait(w1, 1)
        wwait(w1 - 1, 0)
        gstart(w1 + 1, 0)
        wstart(w1, 1)
        w2 = 2 * g + 2
        gwait(w2, 0)
        wwait(w2 - 1, 1)
        gstart(w2 + 1, 1)
        wstart(w2, 0)
        return ()

    lax.fori_loop(0, (nwin - 2) // 2, body, (), unroll=False)
    wl = nwin - 1
    gwait(wl, 1)
    wwait(wl - 1, 0)
    wstart(wl, 1)
    wwait(wl, 1)


def _scatter_msg_body(receivers, msg2d, zeros128, part_out,
                      i0, i1, mbuf, agg_tbl, il0, il1, ls0, ls1, ss0, ss1):
    n_edge = receivers.shape[0]
    per_w = n_edge // NW
    nwin = per_w // W
    core = lax.axis_index("c")
    sid = lax.axis_index("s")
    base = core * (n_edge // 2) + sid * per_w
    rows = N_PAD // 16

    pltpu.sync_copy(zeros128.at[pl.ds(sid * rows, rows)],
                    agg_tbl.at[pl.ds(sid * rows, rows)])
    plsc.subcore_barrier()

    ib = (i0, i1)
    ilsem = (il0, il1)
    lsem = (ls0, ls1)
    ssem = (ss0, ss1)

    def lstart(w, b):
        off = base + w * W
        pltpu.async_copy(receivers.at[pl.ds(off, W)], ib[b], ilsem[b])
        pltpu.async_copy(msg2d.at[pl.ds(off, W)], mbuf.at[b], lsem[b])

    def lwait(w, b):
        off = base + w * W
        pltpu.make_async_copy(receivers.at[pl.ds(off, W)], ib[b],
                              ilsem[b]).wait()
        pltpu.make_async_copy(msg2d.at[pl.ds(off, W)], mbuf.at[b],
                              lsem[b]).wait()

    def sstart(w, b):
        pltpu.async_copy(mbuf.at[b], agg_tbl.at[ib[b]], ssem[b], add=True)

    def swait(w, b):
        pltpu.make_async_copy(mbuf.at[b], agg_tbl.at[ib[b]],
                              ssem[b]).wait()

    lstart(0, 0)
    lwait(0, 0)
    lstart(1, 1)
    sstart(0, 0)

    def body(g, _):
        w1 = 2 * g + 1
        lwait(w1, 1)
        swait(w1 - 1, 0)
        lstart(w1 + 1, 0)
        sstart(w1, 1)
        w2 = 2 * g + 2
        lwait(w2, 0)
        swait(w2 - 1, 1)
        lstart(w2 + 1, 1)
        sstart(w2, 0)
        return ()

    lax.fori_loop(0, (nwin - 2) // 2, body, (), unroll=False)
    wl = nwin - 1
    lwait(wl, 1)
    swait(wl - 1, 0)
    sstart(wl, 1)
    swait(wl, 1)

    plsc.subcore_barrier()
    pltpu.sync_copy(agg_tbl.at[pl.ds(sid * rows, rows)],
                    part_out.at[core, pl.ds(sid * rows, rows)])


def kernel(nodes, senders, receivers, W1, b1, W2, b2):
    n = nodes.shape[0]
    e = senders.shape[0]
    chunk = NW * W * 10
    e_pad = ((e + chunk - 1) // chunk) * chunk
    npad_edges = e_pad - e

    nodes_p = jnp.concatenate(
        [nodes, jnp.zeros((N_PAD - n, D_FEAT), jnp.float32)])
    senders_p = jnp.concatenate(
        [senders, jnp.zeros((npad_edges,), jnp.int32)])
    dummy = n + (jnp.arange(npad_edges, dtype=jnp.int32) % (N_PAD - n))
    receivers_p = jnp.concatenate([receivers, dummy])

    sel = jnp.repeat(jnp.eye(NUM_HEADS, dtype=jnp.float32),
                     FEATS_PER_HEAD, axis=0)
    selt = sel.T

    nblk = 8
    brows = N_PAD // nblk
    emb = pl.pallas_call(
        _mlp_body,
        grid=(nblk,),
        in_specs=[
            pl.BlockSpec((brows, D_FEAT), lambda i: (i, 0)),
            pl.BlockSpec((D_FEAT, D_FEAT), lambda i: (0, 0)),
            pl.BlockSpec((1, D_FEAT), lambda i: (0, 0)),
            pl.BlockSpec((D_FEAT, D_FEAT), lambda i: (0, 0)),
            pl.BlockSpec((1, D_FEAT), lambda i: (0, 0)),
        ],
        out_specs=pl.BlockSpec((brows, D_FEAT), lambda i: (i, 0)),
        out_shape=jax.ShapeDtypeStruct((N_PAD, D_FEAT), jnp.float32),
    )(nodes_p, W1, b1.reshape(1, -1), W2, b2.reshape(1, -1))

    mesh = plsc.VectorSubcoreMesh(core_axis_name="c", subcore_axis_name="s")
    per_w = e_pad // NW

    sent, recv = pl.kernel(
        _gather_body,
        out_type=[jax.ShapeDtypeStruct((e_pad, D_FEAT), jnp.float32),
                  jax.ShapeDtypeStruct((e_pad, D_FEAT), jnp.float32)],
        mesh=mesh,
        scratch_types=[
            pltpu.VMEM((per_w,), jnp.int32),
            pltpu.VMEM((per_w,), jnp.int32),
            pltpu.VMEM((2, W, D_FEAT), jnp.float32),
            pltpu.VMEM((2, W, D_FEAT), jnp.float32),
        ] + [pltpu.SemaphoreType.DMA] * 8,
    )(emb, senders_p, receivers_p)

    be = 2048
    ne_blk = e_pad // be
    logits, bmax = pl.pallas_call(
        _logits_body,
        grid=(ne_blk,),
        in_specs=[
            pl.BlockSpec((be, D_FEAT), lambda i: (i, 0)),
            pl.BlockSpec((be, D_FEAT), lambda i: (i, 0)),
            pl.BlockSpec((D_FEAT, NUM_HEADS), lambda i: (0, 0)),
        ],
        out_specs=[
            pl.BlockSpec((be, NUM_HEADS), lambda i: (i, 0)),
            pl.BlockSpec((1, 1, NUM_HEADS), lambda i: (i, 0, 0)),
        ],
        out_shape=[jax.ShapeDtypeStruct((e_pad, NUM_HEADS), jnp.float32),
                   jax.ShapeDtypeStruct((ne_blk, 1, NUM_HEADS), jnp.float32)],
    )(sent, recv, sel)

    ex = pl.pallas_call(
        _exp_body,
        grid=(ne_blk,),
        in_specs=[
            pl.BlockSpec((be, NUM_HEADS), lambda i: (i, 0)),
            pl.BlockSpec((ne_blk, 1, NUM_HEADS), lambda i: (0, 0, 0)),
        ],
        out_specs=pl.BlockSpec((be, NUM_HEADS), lambda i: (i, 0)),
        out_shape=jax.ShapeDtypeStruct((e_pad, NUM_HEADS), jnp.float32),
    )(logits, bmax)

    msg = pl.pallas_call(
        _msg_body,
        grid=(ne_blk,),
        in_specs=[
            pl.BlockSpec((be, D_FEAT), lambda i: (i, 0)),
            pl.BlockSpec((be, NUM_HEADS), lambda i: (i, 0)),
            pl.BlockSpec((NUM_HEADS, D_FEAT), lambda i: (0, 0)),
        ],
        out_specs=[pl.BlockSpec((be, D_FEAT), lambda i: (i, 0)),
                   pl.BlockSpec((be, D_FEAT), lambda i: (i, 0))],
        out_shape=[jax.ShapeDtypeStruct((e_pad, D_FEAT), jnp.float32),
                   jax.ShapeDtypeStruct((e_pad, D_FEAT), jnp.float32)],
    )(sent, ex, selt)
    msg, wexp = msg

    zeros128 = jnp.zeros((N_PAD, D_FEAT), jnp.float32)
    part = pl.kernel(
        _scatter_msg_body,
        out_type=jax.ShapeDtypeStruct((2, N_PAD, D_FEAT), jnp.float32),
        mesh=mesh,
        scratch_types=[
            pltpu.VMEM((W,), jnp.int32),
            pltpu.VMEM((W,), jnp.int32),
            pltpu.VMEM((2, W, D_FEAT), jnp.float32),
            pltpu.VMEM_SHARED((N_PAD, D_FEAT), jnp.float32),
        ] + [pltpu.SemaphoreType.DMA] * 6,
    )(receivers_p, msg, zeros128)

    segp = pl.kernel(
        _scatter_msg_body,
        out_type=jax.ShapeDtypeStruct((2, N_PAD, D_FEAT), jnp.float32),
        mesh=mesh,
        scratch_types=[
            pltpu.VMEM((W,), jnp.int32),
            pltpu.VMEM((W,), jnp.int32),
            pltpu.VMEM((2, W, D_FEAT), jnp.float32),
            pltpu.VMEM_SHARED((N_PAD, D_FEAT), jnp.float32),
        ] + [pltpu.SemaphoreType.DMA] * 6,
    )(receivers_p, wexp, zeros128)

    nfb = 10
    frows = n // nfb
    out = pl.pallas_call(
        _final_body,
        grid=(nfb,),
        in_specs=[
            pl.BlockSpec((frows, D_FEAT), lambda i: (i, 0)),
            pl.BlockSpec((frows, D_FEAT), lambda i: (i, 0)),
            pl.BlockSpec((frows, D_FEAT), lambda i: (i, 0)),
            pl.BlockSpec((frows, D_FEAT), lambda i: (i, 0)),
        ],
        out_specs=pl.BlockSpec((frows, D_FEAT), lambda i: (i, 0)),
        out_shape=jax.ShapeDtypeStruct((n, D_FEAT), jnp.float32),
    )(part[0], part[1], segp[0], segp[1])

    return out
